# baseline (speedup 1.0000x reference)
import jax
import jax.numpy as jnp
from jax import lax
from jax.experimental import pallas as pl
from jax.experimental.pallas import tpu as pltpu
from jax.experimental.pallas import tpu_sc as plsc

NUM_USERS = 1000000
NUM_ITEMS = 1000000
EMBED = 64
BATCH = 16384

_INFO = plsc.get_sparse_core_info()
_NC = _INFO.num_cores
_NS = _INFO.num_subcores
_NW = _NC * _NS
_BPW = BATCH // _NW
_GCHUNK = 128
_NG = _BPW // _GCHUNK
_L = 16


def _body(user_ids, item_ids, user_table, item_table, out_hbm,
          uidx_v, iidx_v, urows_v, irows_v, tbuf_v, out_v, sem):
    wid = lax.axis_index("s") * _NC + lax.axis_index("c")
    base = wid * _BPW

    for j in range(_NG):
        pltpu.sync_copy(user_ids.at[pl.ds(base + j * _GCHUNK, _GCHUNK)],
                        uidx_v.at[j])
        pltpu.sync_copy(item_ids.at[pl.ds(base + j * _GCHUNK, _GCHUNK)],
                        iidx_v.at[j])

    copies = []
    for j in range(_NG):
        copies.append(pltpu.async_copy(
            user_table.at[uidx_v.at[j]],
            urows_v.at[pl.ds(j * _GCHUNK, _GCHUNK)], sem))
        copies.append(pltpu.async_copy(
            item_table.at[iidx_v.at[j]],
            irows_v.at[pl.ds(j * _GCHUNK, _GCHUNK)], sem))
    for c in copies:
        c.wait()

    nvec = EMBED // _L
    lane = lax.iota(jnp.int32, _L)

    def chunk_body(c, carry):
        for r in range(_L):
            row = c * _L + r
            t = urows_v[row, pl.ds(0, _L)] * irows_v[row, pl.ds(0, _L)]
            for j in range(1, nvec):
                t = t + urows_v[row, pl.ds(j * _L, _L)] * \
                    irows_v[row, pl.ds(j * _L, _L)]
            tbuf_v[r, pl.ds(0, _L)] = t
        acc = plsc.load_gather(tbuf_v, [lane, jnp.zeros((_L,), jnp.int32)])
        for cc in range(1, _L):
            acc = acc + plsc.load_gather(
                tbuf_v, [lane, jnp.full((_L,), cc, jnp.int32)])
        out_v[pl.ds(c * _L, _L)] = acc
        return carry

    lax.fori_loop(0, _BPW // _L, chunk_body, 0)

    pltpu.sync_copy(out_v, out_hbm.at[pl.ds(base, _BPW)])


@jax.jit
def _run(user_ids, item_ids, user_table, item_table):
    mesh = plsc.VectorSubcoreMesh(core_axis_name="c", subcore_axis_name="s")
    f = pl.kernel(
        _body,
        mesh=mesh,
        out_type=jax.ShapeDtypeStruct((BATCH,), jnp.float32),
        scratch_types=[
            pltpu.VMEM((_NG, _GCHUNK), jnp.int32),
            pltpu.VMEM((_NG, _GCHUNK), jnp.int32),
            pltpu.VMEM((_BPW, EMBED), jnp.float32),
            pltpu.VMEM((_BPW, EMBED), jnp.float32),
            pltpu.VMEM((_L, _L + 1), jnp.float32),
            pltpu.VMEM((_BPW,), jnp.float32),
            pltpu.SemaphoreType.DMA,
        ],
        compiler_params=pltpu.CompilerParams(
            needs_layout_passes=False, use_tc_tiling_on_sc=False),
    )
    return f(user_ids, item_ids, user_table, item_table)


def kernel(user_ids, item_ids, user_table, item_table):
    out = _run(user_ids, item_ids, user_table, item_table)
    return out.reshape(BATCH, 1)


# speedup vs baseline: 1.5634x; 1.5634x over previous
import jax
import jax.numpy as jnp
from jax import lax
from jax.experimental import pallas as pl
from jax.experimental.pallas import tpu as pltpu
from jax.experimental.pallas import tpu_sc as plsc

NUM_USERS = 1000000
NUM_ITEMS = 1000000
EMBED = 64
BATCH = 16384

_INFO = plsc.get_sparse_core_info()
_NC = _INFO.num_cores
_NS = _INFO.num_subcores
_NW = _NC * _NS
_BPW = BATCH // _NW
_L = 16
_RPI = 32
_NP = 2
_RPP = _BPW // _NP


def _body(user_ids, item_ids, user_table, item_table, out_hbm,
          u_sm, i_sm, urows_v, irows_v, tbuf_v, out_v, sem):
    wid = lax.axis_index("s") * _NC + lax.axis_index("c")
    base = wid * _BPW

    pltpu.sync_copy(user_ids.at[pl.ds(base, _BPW)], u_sm)
    pltpu.sync_copy(item_ids.at[pl.ds(base, _BPW)], i_sm)

    nvec = EMBED // _L
    lane = lax.iota(jnp.int32, _L)

    for p in range(_NP):
        pbase = p * _RPP

        def fire_body(g, carry):
            for v in range(_RPI // _L):
                lr0 = g * _RPI + v * _L
                uvec = u_sm[pl.ds(pbase + lr0, _L)]
                ivec = i_sm[pl.ds(pbase + lr0, _L)]
                for r in range(_L):
                    lr = lr0 + r
                    pltpu.async_copy(user_table.at[pl.ds(uvec[r], 1)],
                                     urows_v.at[pl.ds(lr, 1)], sem)
                    pltpu.async_copy(item_table.at[pl.ds(ivec[r], 1)],
                                     irows_v.at[pl.ds(lr, 1)], sem)
            return carry

        lax.fori_loop(0, _RPP // _RPI, fire_body, 0)

        def drain_body(g, carry):
            for r in range(_RPI):
                lr = g * _RPI + r
                pltpu.make_async_copy(user_table.at[pl.ds(0, 1)],
                                      urows_v.at[pl.ds(lr, 1)], sem).wait()
                pltpu.make_async_copy(item_table.at[pl.ds(0, 1)],
                                      irows_v.at[pl.ds(lr, 1)], sem).wait()
            return carry

        lax.fori_loop(0, _RPP // _RPI, drain_body, 0)

        def chunk_body(c, carry):
            for r in range(_L):
                row = c * _L + r
                t = urows_v[row, pl.ds(0, _L)] * irows_v[row, pl.ds(0, _L)]
                for j in range(1, nvec):
                    t = t + urows_v[row, pl.ds(j * _L, _L)] * \
                        irows_v[row, pl.ds(j * _L, _L)]
                tbuf_v[r, pl.ds(0, _L)] = t
            acc = plsc.load_gather(
                tbuf_v, [lane, jnp.zeros((_L,), jnp.int32)])
            for cc in range(1, _L):
                acc = acc + plsc.load_gather(
                    tbuf_v, [lane, jnp.full((_L,), cc, jnp.int32)])
            out_v[pl.ds(pbase + c * _L, _L)] = acc
            return carry

        lax.fori_loop(0, _RPP // _L, chunk_body, 0)

    pltpu.sync_copy(out_v, out_hbm.at[pl.ds(base, _BPW)])


@jax.jit
def _run(user_ids, item_ids, user_table, item_table):
    mesh = plsc.VectorSubcoreMesh(core_axis_name="c", subcore_axis_name="s")
    f = pl.kernel(
        _body,
        mesh=mesh,
        out_type=jax.ShapeDtypeStruct((BATCH,), jnp.float32),
        scratch_types=[
            pltpu.VMEM((_BPW,), jnp.int32),
            pltpu.VMEM((_BPW,), jnp.int32),
            pltpu.VMEM((_RPP, EMBED), jnp.float32),
            pltpu.VMEM((_RPP, EMBED), jnp.float32),
            pltpu.VMEM((_L, _L + 1), jnp.float32),
            pltpu.VMEM((_BPW,), jnp.float32),
            pltpu.SemaphoreType.DMA,
        ],
        compiler_params=pltpu.CompilerParams(needs_layout_passes=False),
    )
    return f(user_ids, item_ids, user_table, item_table)


def kernel(user_ids, item_ids, user_table, item_table):
    out = _run(user_ids, item_ids, user_table, item_table)
    return out.reshape(BATCH, 1)
